# Initial kernel scaffold; baseline (speedup 1.0000x reference)
import jax
import jax.numpy as jnp
from jax.experimental import pallas as pl


def kernel(index, table):
    raise NotImplementedError("write your pallas kernel here")


# speedup vs baseline: 1.0916x; 1.0916x over previous
import functools
import math

import jax
import jax.numpy as jnp
import numpy as np
from jax import lax
from jax.experimental import pallas as pl
from jax.experimental.pallas import tpu as pltpu
from jax.experimental.pallas import tpu_sc as plsc

EMBED_NUM = 1000000
HIDDEN_DIM = 32
LANES = 16
PI = np.float32(math.pi)
HALF = np.float32(EMBED_NUM // 2)
ONE = np.float32(1.0)

NUM_CORES = 2
NUM_SUBCORES = 16
NUM_WORKERS = NUM_CORES * NUM_SUBCORES

CHUNK = 1024
GATHER = 128
G_PER_CHUNK = CHUNK // GATHER


def _sc_embed(index_flat, table, *, total):
    b_per_w = total // NUM_WORKERS
    n_chunks = b_per_w // CHUNK
    mesh = plsc.VectorSubcoreMesh(core_axis_name="c", subcore_axis_name="s")

    @functools.partial(
        pl.kernel,
        mesh=mesh,
        out_type=jax.ShapeDtypeStruct((total, HIDDEN_DIM), jnp.float32),
        scratch_types=[
            pltpu.VMEM((CHUNK,), jnp.float32),
            pltpu.VMEM((G_PER_CHUNK, GATHER), jnp.int32),
            pltpu.VMEM((CHUNK, HIDDEN_DIM), jnp.float32),
            pltpu.SemaphoreType.DMA,
        ],
        compiler_params=pltpu.CompilerParams(use_tc_tiling_on_sc=False),
    )
    def body(ang_hbm, table_hbm, out_hbm, ang_v, idx_v, rows_v, sem):
        wid = lax.axis_index("s") * NUM_CORES + lax.axis_index("c")
        base = wid * b_per_w

        def chunk_body(ci, _):
            off = base + ci * CHUNK
            pltpu.sync_copy(ang_hbm.at[pl.ds(off, CHUNK)], ang_v)
            for j in range(G_PER_CHUNK):
                for i in range(GATHER // LANES):
                    x = ang_v[pl.ds((j * (GATHER // LANES) + i) * LANES, LANES)]
                    y = (x / PI + ONE) * HALF
                    ii = y.astype(jnp.int32)
                    ii = jnp.minimum(jnp.maximum(ii, 0), EMBED_NUM - 1)
                    idx_v[j, pl.ds(i * LANES, LANES)] = ii
            copies = []
            for j in range(G_PER_CHUNK):
                copies.append(
                    pltpu.make_async_copy(
                        table_hbm.at[idx_v.at[j]],
                        rows_v.at[pl.ds(j * GATHER, GATHER)],
                        sem,
                    )
                )
                copies[-1].start()
            for c in copies:
                c.wait()
            pltpu.sync_copy(rows_v, out_hbm.at[pl.ds(off, CHUNK)])
            return 0

        lax.fori_loop(0, n_chunks, chunk_body, 0)

    return body(index_flat, table)


def kernel(index, table):
    total = index.shape[0] * index.shape[1]
    flat = index.reshape(total)
    out = _sc_embed(flat, table, total=total)
    return out.reshape(index.shape[0], index.shape[1], HIDDEN_DIM)


# speedup vs baseline: 1.1106x; 1.0174x over previous
import functools
import math

import jax
import jax.numpy as jnp
import numpy as np
from jax import lax
from jax.experimental import pallas as pl
from jax.experimental.pallas import tpu as pltpu
from jax.experimental.pallas import tpu_sc as plsc

EMBED_NUM = 1000000
HIDDEN_DIM = 32
LANES = 16
PI = np.float32(math.pi)
HALF = np.float32(EMBED_NUM // 2)
ONE = np.float32(1.0)

NUM_CORES = 2
NUM_SUBCORES = 16
NUM_WORKERS = NUM_CORES * NUM_SUBCORES

CHUNK = 512
GATHER = 128
G_PER_CHUNK = CHUNK // GATHER
NBUF = 2


def _sc_embed(index_flat, table, *, total):
    b_per_w = total // NUM_WORKERS
    n_chunks = b_per_w // CHUNK
    assert n_chunks % NBUF == 0
    mesh = plsc.VectorSubcoreMesh(core_axis_name="c", subcore_axis_name="s")

    @functools.partial(
        pl.kernel,
        mesh=mesh,
        out_type=jax.ShapeDtypeStruct((total, HIDDEN_DIM), jnp.float32),
        scratch_types=[
            pltpu.VMEM((NBUF, CHUNK), jnp.float32),
            pltpu.VMEM((NBUF * G_PER_CHUNK, GATHER), jnp.int32),
            pltpu.VMEM((NBUF, CHUNK, HIDDEN_DIM), jnp.float32),
            pltpu.SemaphoreType.DMA,
            pltpu.SemaphoreType.DMA,
            pltpu.SemaphoreType.DMA,
            pltpu.SemaphoreType.DMA,
        ],
        compiler_params=pltpu.CompilerParams(use_tc_tiling_on_sc=False),
    )
    def body(ang_hbm, table_hbm, out_hbm, ang_v, idx_v, rows_v, sg0, sg1,
             ss0, ss1):
        wid = lax.axis_index("s") * NUM_CORES + lax.axis_index("c")
        base = wid * b_per_w
        sem_g = (sg0, sg1)
        sem_st = (ss0, ss1)

        def gather_copies(b):
            return [
                pltpu.make_async_copy(
                    table_hbm.at[idx_v.at[b * G_PER_CHUNK + j]],
                    rows_v.at[b, pl.ds(j * GATHER, GATHER)],
                    sem_g[b],
                )
                for j in range(G_PER_CHUNK)
            ]

        def store_copy(b, ci):
            return pltpu.make_async_copy(
                rows_v.at[b],
                out_hbm.at[pl.ds(base + ci * CHUNK, CHUNK)],
                sem_st[b],
            )

        def outer(gi, _):
            for b in range(NBUF):
                ci = gi * NBUF + b
                pb = 1 - b
                pltpu.sync_copy(ang_hbm.at[pl.ds(base + ci * CHUNK, CHUNK)],
                                ang_v.at[b])
                for j in range(G_PER_CHUNK):
                    for i in range(GATHER // LANES):
                        x = ang_v[b, pl.ds((j * (GATHER // LANES) + i) * LANES,
                                           LANES)]
                        y = (x / PI + ONE) * HALF
                        ii = y.astype(jnp.int32)
                        ii = jnp.minimum(jnp.maximum(ii, 0), EMBED_NUM - 1)
                        idx_v[b * G_PER_CHUNK + j, pl.ds(i * LANES, LANES)] = ii
                @pl.when(ci >= NBUF)
                def _():
                    store_copy(b, ci - NBUF).wait()
                for c in gather_copies(b):
                    c.start()
                @pl.when(ci >= 1)
                def _():
                    for c in gather_copies(pb):
                        c.wait()
                    store_copy(pb, ci - 1).start()
            return 0

        lax.fori_loop(0, n_chunks // NBUF, outer, 0)
        last = n_chunks - 1
        lb = last % NBUF
        for c in gather_copies(lb):
            c.wait()
        store_copy(lb, last).start()
        store_copy(1 - lb, last - 1).wait()
        store_copy(lb, last).wait()

    return body(index_flat, table)


def kernel(index, table):
    total = index.shape[0] * index.shape[1]
    flat = index.reshape(total)
    out = _sc_embed(flat, table, total=total)
    return out.reshape(index.shape[0], index.shape[1], HIDDEN_DIM)


# speedup vs baseline: 1.7967x; 1.6178x over previous
import functools
import math

import jax
import jax.numpy as jnp
import numpy as np
from jax import lax
from jax.experimental import pallas as pl
from jax.experimental.pallas import tpu as pltpu
from jax.experimental.pallas import tpu_sc as plsc

EMBED_NUM = 1000000
HIDDEN_DIM = 32
LANES = 16
PI = np.float32(math.pi)
HALF = np.float32(EMBED_NUM // 2)
ONE = np.float32(1.0)

NUM_CORES = 2
NUM_SUBCORES = 16
NUM_WORKERS = NUM_CORES * NUM_SUBCORES

SEQ = 50
SAMP_PER_CHUNK = 8
CHUNK = SAMP_PER_CHUNK * SEQ
G_IDX = 100
G_PER_CHUNK = CHUNK // G_IDX
NBUF = 2


def _sc_embed(index_flat, table, *, n_samples):
    samp_per_w = n_samples // NUM_WORKERS
    n_chunks = samp_per_w // SAMP_PER_CHUNK
    assert n_chunks % NBUF == 0
    mesh = plsc.VectorSubcoreMesh(core_axis_name="c", subcore_axis_name="s")

    @functools.partial(
        pl.kernel,
        mesh=mesh,
        out_type=jax.ShapeDtypeStruct((n_samples, SEQ, HIDDEN_DIM),
                                      jnp.float32),
        scratch_types=[
            pltpu.VMEM((NBUF, CHUNK), jnp.float32),
            pltpu.VMEM((NBUF * G_PER_CHUNK, G_IDX), jnp.int32),
            pltpu.VMEM((NBUF, CHUNK, HIDDEN_DIM), jnp.float32),
            pltpu.SemaphoreType.DMA,
            pltpu.SemaphoreType.DMA,
            pltpu.SemaphoreType.DMA,
            pltpu.SemaphoreType.DMA,
        ],
        compiler_params=pltpu.CompilerParams(use_tc_tiling_on_sc=False),
    )
    def body(ang_hbm, table_hbm, out_hbm, ang_v, idx_v, rows_v, sg0, sg1,
             ss0, ss1):
        wid = lax.axis_index("s") * NUM_CORES + lax.axis_index("c")
        samp_base = wid * samp_per_w
        sem_g = (sg0, sg1)
        sem_st = (ss0, ss1)

        def gather_copies(b):
            return [
                pltpu.make_async_copy(
                    table_hbm.at[idx_v.at[b * G_PER_CHUNK + j]],
                    rows_v.at[b, pl.ds(j * G_IDX, G_IDX)],
                    sem_g[b],
                )
                for j in range(G_PER_CHUNK)
            ]

        def store_copies(b, ci):
            s0 = samp_base + ci * SAMP_PER_CHUNK
            return [
                pltpu.make_async_copy(
                    rows_v.at[b, pl.ds(s * SEQ, SEQ)],
                    out_hbm.at[s0 + s],
                    sem_st[b],
                )
                for s in range(SAMP_PER_CHUNK)
            ]

        def compute_idx(b, ci):
            off = (samp_base + ci * SAMP_PER_CHUNK) * SEQ
            pltpu.sync_copy(ang_hbm.at[pl.ds(off, CHUNK)], ang_v.at[b])
            for j in range(G_PER_CHUNK):
                starts = [i * LANES for i in range(G_IDX // LANES)]
                starts.append(G_IDX - LANES)
                for s in starts:
                    x = ang_v[b, pl.ds(j * G_IDX + s, LANES)]
                    y = (x / PI + ONE) * HALF
                    ii = y.astype(jnp.int32)
                    ii = jnp.minimum(jnp.maximum(ii, 0), EMBED_NUM - 1)
                    idx_v[b * G_PER_CHUNK + j, pl.ds(s, LANES)] = ii

        def outer(gi, _):
            for b in range(NBUF):
                ci = gi * NBUF + b
                pb = 1 - b
                compute_idx(b, ci)
                @pl.when(ci >= NBUF)
                def _():
                    for c in store_copies(b, ci - NBUF):
                        c.wait()
                for c in gather_copies(b):
                    c.start()
                @pl.when(ci >= 1)
                def _():
                    for c in gather_copies(pb):
                        c.wait()
                    for c in store_copies(pb, ci - 1):
                        c.start()
            return 0

        lax.fori_loop(0, n_chunks // NBUF, outer, 0)
        last = n_chunks - 1
        lb = last % NBUF
        for c in gather_copies(lb):
            c.wait()
        for c in store_copies(lb, last):
            c.start()
        for c in store_copies(1 - lb, last - 1):
            c.wait()
        for c in store_copies(lb, last):
            c.wait()

    return body(index_flat, table)


def kernel(index, table):
    n_samples = index.shape[0]
    flat = index.reshape(n_samples * SEQ)
    return _sc_embed(flat, table, n_samples=n_samples)
